# baseline (speedup 1.0000x reference)
import functools

import jax
import jax.numpy as jnp
import numpy as np
from jax import lax
from jax.experimental import pallas as pl
from jax.experimental.pallas import tpu as pltpu
from jax.experimental.pallas import tpu_sc as plsc

D_MODEL = 128
VOCAB = 1000
SCALE = np.sqrt(np.float32(D_MODEL)).astype(np.float32)

NUM_CORES = 2
NUM_SUBCORES = 16
NUM_WORKERS = NUM_CORES * NUM_SUBCORES
IW = 128
GPC = 2
CHUNK = IW * GPC
LANES = 16
ROWS_A = 64


def _embed_kernel(n_rows: int):
    rows_per_worker = n_rows // NUM_WORKERS
    n_chunks = rows_per_worker // CHUNK
    n_irows = rows_per_worker // IW
    mesh = plsc.VectorSubcoreMesh(core_axis_name="c", subcore_axis_name="s")

    @functools.partial(
        pl.kernel,
        out_type=jax.ShapeDtypeStruct((n_rows, D_MODEL), jnp.float32),
        mesh=mesh,
        compiler_params=pltpu.CompilerParams(use_tc_tiling_on_sc=True),
        scratch_types=[
            pltpu.VMEM_SHARED((VOCAB, D_MODEL), jnp.float32),
            pltpu.VMEM((n_irows, IW), jnp.int32),
            pltpu.VMEM((CHUNK, D_MODEL), jnp.float32),
            pltpu.VMEM((CHUNK, D_MODEL), jnp.float32),
            pltpu.SemaphoreType.DMA,
            pltpu.SemaphoreType.DMA,
            pltpu.SemaphoreType.DMA,
            pltpu.SemaphoreType.DMA,
            pltpu.SemaphoreType.DMA,
        ],
    )
    def k(x_hbm, table_hbm, out_hbm, tbl_sh, idx_v, rows0, rows1,
          gsem0, gsem1, ssem0, ssem1, isem):
        cid = lax.axis_index("c")
        sid = lax.axis_index("s")
        wid = sid * NUM_CORES + cid
        wbase = wid * rows_per_worker
        rows = (rows0, rows1)
        gsem = (gsem0, gsem1)
        ssem = (ssem0, ssem1)

        idx_copy = pltpu.async_copy(
            x_hbm.at[pl.ds(wid * n_irows, n_irows)], idx_v, isem)

        abase = jnp.minimum(sid * ROWS_A, VOCAB - ROWS_A)
        stage = rows0.at[pl.ds(0, ROWS_A)]
        pltpu.sync_copy(table_hbm.at[pl.ds(abase, ROWS_A)], stage)

        def scale_body(i, c):
            for jj in range(D_MODEL // LANES):
                sl = pl.ds(jj * LANES, LANES)
                rows0[i, sl] = rows0[i, sl] * SCALE
            return c

        lax.fori_loop(0, ROWS_A, scale_body, 0)
        pltpu.sync_copy(stage, tbl_sh.at[pl.ds(abase, ROWS_A)])
        idx_copy.wait()
        plsc.subcore_barrier()

        def fire_gather(j, b):
            for g in range(GPC):
                pltpu.async_copy(
                    tbl_sh.at[idx_v.at[j * GPC + g]],
                    rows[b].at[pl.ds(g * IW, IW)], gsem[b])

        def wait_gather(b):
            for _ in range(GPC):
                pltpu.make_async_copy(
                    tbl_sh.at[idx_v.at[0]],
                    rows[b].at[pl.ds(0, IW)], gsem[b]).wait()

        def fire_store(j, b):
            pltpu.async_copy(
                rows[b], out_hbm.at[pl.ds(wbase + j * CHUNK, CHUNK)], ssem[b])

        def wait_store(b):
            pltpu.make_async_copy(
                rows[b], out_hbm.at[pl.ds(wbase, CHUNK)], ssem[b]).wait()

        fire_gather(0, 0)
        fire_gather(1, 1)
        wait_gather(0)
        fire_store(0, 0)
        wait_store(0)
        fire_gather(2, 0)
        wait_gather(1)
        fire_store(1, 1)

        def pair_body(j2, c):
            j = 2 * j2
            wait_store(1)
            fire_gather(j + 1, 1)
            wait_gather(0)
            fire_store(j, 0)
            wait_store(0)
            fire_gather(j + 2, 0)
            wait_gather(1)
            fire_store(j + 1, 1)
            return c

        lax.fori_loop(1, n_chunks // 2 - 1, pair_body, 0)

        wait_store(1)
        fire_gather(n_chunks - 1, 1)
        wait_gather(0)
        fire_store(n_chunks - 2, 0)
        wait_gather(1)
        fire_store(n_chunks - 1, 1)
        wait_store(0)
        wait_store(1)

    return k


@jax.jit
def kernel(x, table):
    b, h = x.shape
    n_rows = b * h
    flat = x.T.reshape(n_rows // IW, IW)
    out = _embed_kernel(n_rows)(flat, table)
    return out.reshape(h, b, D_MODEL).transpose(1, 0, 2)


# speedup vs baseline: 1.0358x; 1.0358x over previous
import functools

import jax
import jax.numpy as jnp
import numpy as np
from jax import lax
from jax.experimental import pallas as pl
from jax.experimental.pallas import tpu as pltpu
from jax.experimental.pallas import tpu_sc as plsc

D_MODEL = 128
VOCAB = 1000
SCALE = np.sqrt(np.float32(D_MODEL)).astype(np.float32)

NUM_CORES = 2
NUM_SUBCORES = 16
NUM_WORKERS = NUM_CORES * NUM_SUBCORES
CHUNK = 128
LANES = 16
ROWS_A = 64
NBUF = 4
SUBS = 4


def _embed_kernel(batch: int, hist: int):
    cols_per_w = batch // NUM_WORKERS
    mesh = plsc.VectorSubcoreMesh(core_axis_name="c", subcore_axis_name="s")

    @functools.partial(
        pl.kernel,
        out_type=jax.ShapeDtypeStruct((batch * hist, D_MODEL), jnp.float32),
        mesh=mesh,
        compiler_params=pltpu.CompilerParams(use_tc_tiling_on_sc=True),
        scratch_types=[
            pltpu.VMEM_SHARED((VOCAB, D_MODEL), jnp.float32),
            pltpu.VMEM((hist, cols_per_w), jnp.int32),
        ]
        + [pltpu.VMEM((CHUNK, D_MODEL), jnp.float32)] * NBUF
        + [pltpu.SemaphoreType.DMA] * (2 * NBUF + 1),
    )
    def k(x_hbm, table_hbm, out_hbm, tbl_sh, idx_v, r0, r1, r2, r3,
          g0, g1, g2, g3, s0, s1, s2, s3, isem):
        cid = lax.axis_index("c")
        sid = lax.axis_index("s")
        wid = sid * NUM_CORES + cid
        cbase = wid * cols_per_w
        rows = (r0, r1, r2, r3)
        gsem = (g0, g1, g2, g3)
        ssem = (s0, s1, s2, s3)

        idx_copy = pltpu.async_copy(
            x_hbm.at[:, pl.ds(cbase, cols_per_w)], idx_v, isem)

        abase = jnp.minimum(sid * ROWS_A, VOCAB - ROWS_A)
        stage = r0.at[pl.ds(0, ROWS_A)]
        pltpu.sync_copy(table_hbm.at[pl.ds(abase, ROWS_A)], stage)

        def scale_body(i, c):
            for jj in range(D_MODEL // LANES):
                sl = pl.ds(jj * LANES, LANES)
                r0[i, sl] = r0[i, sl] * SCALE
            return c

        lax.fori_loop(0, ROWS_A, scale_body, 0)
        pltpu.sync_copy(stage, tbl_sh.at[pl.ds(abase, ROWS_A)])
        idx_copy.wait()
        plsc.subcore_barrier()

        def fire_gather(h, sub, b):
            pltpu.async_copy(
                tbl_sh.at[idx_v.at[h, pl.ds(sub * CHUNK, CHUNK)]],
                rows[b], gsem[b])

        def wait_gather(b):
            pltpu.make_async_copy(
                tbl_sh.at[idx_v.at[0, pl.ds(0, CHUNK)]], rows[b],
                gsem[b]).wait()

        def fire_store(h, sub, b):
            pltpu.async_copy(
                rows[b],
                out_hbm.at[pl.ds(h * batch + cbase + sub * CHUNK, CHUNK)],
                ssem[b])

        def wait_store(b):
            pltpu.make_async_copy(
                rows[b], out_hbm.at[pl.ds(cbase, CHUNK)], ssem[b]).wait()

        def step(h, sub, fire_ahead=True, drain=True):
            nb = (sub + 2) % NBUF
            if fire_ahead:
                if drain:
                    wait_store(nb)
                fire_gather(h + (1 if sub >= 2 else 0), nb, nb)
            wait_gather(sub)
            fire_store(h, sub, sub)

        fire_gather(0, 0, 0)
        fire_gather(0, 1, 1)
        step(0, 0, drain=False)
        step(0, 1, drain=False)
        step(0, 2)
        step(0, 3)

        def quad_body(h, c):
            step(h, 0)
            step(h, 1)
            step(h, 2)
            step(h, 3)
            return c

        lax.fori_loop(1, hist - 1, quad_body, 0)

        step(hist - 1, 0)
        step(hist - 1, 1)
        step(hist - 1, 2, fire_ahead=False)
        step(hist - 1, 3, fire_ahead=False)
        for b in range(NBUF):
            wait_store(b)

    return k


@jax.jit
def kernel(x, table):
    b, h = x.shape
    out = _embed_kernel(b, h)(x.T, table)
    return out.reshape(h, b, D_MODEL).transpose(1, 0, 2)
